# baseline (speedup 1.0000x reference)
import jax
import jax.numpy as jnp
from jax import lax
from jax.experimental import pallas as pl
from jax.experimental.pallas import tpu as pltpu
from jax.experimental.pallas import tpu_sc as plsc

B, L, C = 4, 8192, 768
K = L // 2
HB = 2
NSH = 16
SHARD = L // NSH
OUT_T = K // NSH
CH = 64
STAGE_ROW = K + 128


def _tc_body(x_ref, keys_ref, tval_ref, rval_ref):
    x = x_ref[...]
    n = jnp.sqrt(jnp.sum(x * x, axis=2))
    keys_ref[...] = n[:, None, :]
    kb = lax.bitcast_convert_type(keys_ref[...], jnp.int32)

    def body(i, t):
        cand = t | (jnp.int32(1) << (30 - i))
        cnt = jnp.sum((kb >= cand).astype(jnp.int32))
        return jnp.where(cnt >= K, cand, t)

    t_bits = lax.fori_loop(0, 31, body, jnp.int32(0))
    cgt = jnp.sum((kb > t_bits).astype(jnp.int32))
    tval_ref[...] = jnp.full(
        (1, 1, 128), lax.bitcast_convert_type(t_bits, jnp.float32))
    rval_ref[...] = jnp.full((1, 1, 128), K - cgt, jnp.int32)


def _tc_stage(xh):
    keys, tval, rval = pl.pallas_call(
        _tc_body,
        grid=(HB,),
        in_specs=[pl.BlockSpec((1, L, C), lambda b: (b, 0, 0))],
        out_specs=[
            pl.BlockSpec((1, 1, L), lambda b: (b, 0, 0)),
            pl.BlockSpec((1, 1, 128), lambda b: (b, 0, 0)),
            pl.BlockSpec((1, 1, 128), lambda b: (b, 0, 0)),
        ],
        out_shape=[
            jax.ShapeDtypeStruct((HB, 1, L), jnp.float32),
            jax.ShapeDtypeStruct((HB, 1, 128), jnp.float32),
            jax.ShapeDtypeStruct((HB, 1, 128), jnp.int32),
        ],
    )(xh)
    return keys.reshape(HB, L), tval.reshape(HB, 128), rval.reshape(HB, 128)


def _sc_half_body(bbase, x_ref, keys_ref, tval_ref, rval_ref,
                  xs_ref, idx_ref,
                  keys_v, tv, rv, posb, valb, idxmy, rows_a, rows_b,
                  stage_sh, gsem):
    c = lax.axis_index("c")
    s = lax.axis_index("s")
    b = bbase + c
    start = s * SHARD

    pltpu.sync_copy(keys_ref.at[c], keys_v)
    pltpu.sync_copy(tval_ref.at[c], tv)
    pltpu.sync_copy(rval_ref.at[c], rv)
    tvec = tv[pl.ds(0, 16)]
    rvec = rv[pl.ds(0, 16)]
    lane = lax.iota(jnp.int32, 16)

    def pbody(j, carry):
        gt, eq = carry
        kv = keys_v[pl.ds(j * 16, 16)]
        gt = gt + jnp.sum((kv > tvec).astype(jnp.int32))
        eq = eq + jnp.sum((kv == tvec).astype(jnp.int32))
        return gt, eq

    zeros = jnp.zeros((16,), jnp.int32)
    gt0, eq0 = lax.fori_loop(0, s * (SHARD // 16), pbody, (zeros, zeros))

    off = gt0 + jnp.minimum(eq0, rvec)
    eqs = eq0
    for j in range(SHARD // 16):
        kv = keys_v[pl.ds(start + j * 16, 16)]
        gtm = kv > tvec
        eqm = kv == tvec
        eqi = eqm.astype(jnp.int32)
        eq_excl = plsc.cumsum(eqi) - eqi
        sel = gtm | (eqm & ((eqs + eq_excl) < rvec))
        seli = sel.astype(jnp.int32)
        rank = plsc.cumsum(seli) - seli
        dump = K + ((s * 16 + lane) % 128)
        pos = jnp.where(sel, off + rank, dump)
        posb[j // 8, pl.ds((j % 8) * 16, 16)] = pos
        valb[j // 8, pl.ds((j % 8) * 16, 16)] = start + j * 16 + lane
        off = off + jnp.sum(seli)
        eqs = eqs + jnp.sum(eqi)

    for t in range(SHARD // 128):
        pltpu.sync_copy(valb.at[t], stage_sh.at[posb.at[t]])

    plsc.subcore_barrier()

    kbase = s * OUT_T
    pltpu.sync_copy(stage_sh.at[pl.ds(kbase, OUT_T)], idxmy)
    pltpu.sync_copy(idxmy, idx_ref.at[b, pl.ds(kbase, OUT_T)])

    bufs = (rows_a, rows_b)
    desc = pltpu.async_copy(
        x_ref.at[b].at[idxmy.at[pl.ds(0, CH)]], bufs[0], gsem)
    for t in range(OUT_T // CH):
        desc.wait()
        cur = bufs[t % 2]
        if t < OUT_T // CH - 1:
            desc = pltpu.async_copy(
                x_ref.at[b].at[idxmy.at[pl.ds((t + 1) * CH, CH)]],
                bufs[(t + 1) % 2], gsem)
        pltpu.sync_copy(cur, xs_ref.at[b, pl.ds(kbase + t * CH, CH), :])


def _sc_scratch():
    return [
        pltpu.VMEM((L,), jnp.float32),
        pltpu.VMEM((128,), jnp.float32),
        pltpu.VMEM((128,), jnp.int32),
        pltpu.VMEM((SHARD // 128, 128), jnp.int32),
        pltpu.VMEM((SHARD // 128, 128), jnp.int32),
        pltpu.VMEM((OUT_T,), jnp.int32),
        pltpu.VMEM((CH, C), jnp.float32),
        pltpu.VMEM((CH, C), jnp.float32),
        pltpu.VMEM_SHARED((STAGE_ROW,), jnp.int32),
        pltpu.SemaphoreType.DMA,
    ]


def _sc_half0(x, keys, tval, rval):
    mesh = plsc.VectorSubcoreMesh(core_axis_name="c", subcore_axis_name="s")

    def body(*refs):
        _sc_half_body(0, *refs)

    return pl.kernel(
        body,
        (jax.ShapeDtypeStruct((B, K, C), jnp.float32),
         jax.ShapeDtypeStruct((B, K), jnp.int32)),
        mesh=mesh,
        compiler_params=pltpu.CompilerParams(needs_layout_passes=False),
        scratch_types=_sc_scratch(),
    )(x, keys, tval, rval)


def _sc_half1(x, keys, tval, rval, xs_r, idx_r):
    mesh = plsc.VectorSubcoreMesh(core_axis_name="c", subcore_axis_name="s")

    def body(*refs):
        _sc_half_body(2, *refs)

    pl.kernel(
        body,
        (),
        mesh=mesh,
        compiler_params=pltpu.CompilerParams(needs_layout_passes=False),
        scratch_types=_sc_scratch(),
    )(x, keys, tval, rval, xs_r, idx_r)


def kernel(x):
    k0, t0, r0 = _tc_stage(x[0:HB])
    k1, t1, r1 = _tc_stage(x[HB:])
    xs_half, idx_half = _sc_half0(x, k0, t0, r0)
    xs_r = jax.new_ref(xs_half)
    idx_r = jax.new_ref(idx_half)
    _sc_half1(x, k1, t1, r1, xs_r, idx_r)
    return (xs_r[...], idx_r[...])


# speedup vs baseline: 1.5389x; 1.5389x over previous
import jax
import jax.numpy as jnp
from jax import lax
from jax.experimental import pallas as pl
from jax.experimental.pallas import tpu as pltpu
from jax.experimental.pallas import tpu_sc as plsc

B, L, C = 4, 8192, 768
K = L // 2
HB = 2
NSH = 16
SHARD = L // NSH
OUT_T = K // NSH
CH = 64
STAGE_ROW = K + 128


def _tc_body(x_ref, keys_ref, tval_ref, rval_ref):
    x = x_ref[...]
    n = jnp.sqrt(jnp.sum(x * x, axis=2))
    keys_ref[...] = n[:, None, :]
    kb = lax.bitcast_convert_type(keys_ref[...], jnp.int32)

    def body(i, t):
        cand = t | (jnp.int32(1) << (30 - i))
        cnt = jnp.sum((kb >= cand).astype(jnp.int32))
        return jnp.where(cnt >= K, cand, t)

    t_bits = lax.fori_loop(0, 31, body, jnp.int32(0))
    cgt = jnp.sum((kb > t_bits).astype(jnp.int32))
    tval_ref[...] = jnp.full(
        (1, 1, 128), lax.bitcast_convert_type(t_bits, jnp.float32))
    rval_ref[...] = jnp.full((1, 1, 128), K - cgt, jnp.int32)


def _tc_stage(x, bbase):
    keys, tval, rval = pl.pallas_call(
        _tc_body,
        grid=(HB,),
        in_specs=[pl.BlockSpec((1, L, C), lambda b: (bbase + b, 0, 0))],
        out_specs=[
            pl.BlockSpec((1, 1, L), lambda b: (b, 0, 0)),
            pl.BlockSpec((1, 1, 128), lambda b: (b, 0, 0)),
            pl.BlockSpec((1, 1, 128), lambda b: (b, 0, 0)),
        ],
        out_shape=[
            jax.ShapeDtypeStruct((HB, 1, L), jnp.float32),
            jax.ShapeDtypeStruct((HB, 1, 128), jnp.float32),
            jax.ShapeDtypeStruct((HB, 1, 128), jnp.int32),
        ],
    )(x)
    return keys.reshape(HB, L), tval.reshape(HB, 128), rval.reshape(HB, 128)


def _sc_half_body(bbase, x_ref, keys_ref, tval_ref, rval_ref,
                  xs_ref, idx_ref,
                  keys_v, tv, rv, posb, valb, idxmy, rows_a, rows_b,
                  stage_sh, gsem):
    c = lax.axis_index("c")
    s = lax.axis_index("s")
    b = bbase + c
    start = s * SHARD

    pltpu.sync_copy(keys_ref.at[c], keys_v)
    pltpu.sync_copy(tval_ref.at[c], tv)
    pltpu.sync_copy(rval_ref.at[c], rv)
    tvec = tv[pl.ds(0, 16)]
    rvec = rv[pl.ds(0, 16)]
    lane = lax.iota(jnp.int32, 16)

    def pbody(j, carry):
        gt, eq = carry
        kv = keys_v[pl.ds(j * 16, 16)]
        gt = gt + jnp.sum((kv > tvec).astype(jnp.int32))
        eq = eq + jnp.sum((kv == tvec).astype(jnp.int32))
        return gt, eq

    zeros = jnp.zeros((16,), jnp.int32)
    gt0, eq0 = lax.fori_loop(0, s * (SHARD // 16), pbody, (zeros, zeros))

    off = gt0 + jnp.minimum(eq0, rvec)
    eqs = eq0
    for j in range(SHARD // 16):
        kv = keys_v[pl.ds(start + j * 16, 16)]
        gtm = kv > tvec
        eqm = kv == tvec
        eqi = eqm.astype(jnp.int32)
        eq_excl = plsc.cumsum(eqi) - eqi
        sel = gtm | (eqm & ((eqs + eq_excl) < rvec))
        seli = sel.astype(jnp.int32)
        rank = plsc.cumsum(seli) - seli
        dump = K + ((s * 16 + lane) % 128)
        pos = jnp.where(sel, off + rank, dump)
        posb[j // 8, pl.ds((j % 8) * 16, 16)] = pos
        valb[j // 8, pl.ds((j % 8) * 16, 16)] = start + j * 16 + lane
        off = off + jnp.sum(seli)
        eqs = eqs + jnp.sum(eqi)

    for t in range(SHARD // 128):
        pltpu.sync_copy(valb.at[t], stage_sh.at[posb.at[t]])

    plsc.subcore_barrier()

    kbase = s * OUT_T
    pltpu.sync_copy(stage_sh.at[pl.ds(kbase, OUT_T)], idxmy)
    pltpu.sync_copy(idxmy, idx_ref.at[b, pl.ds(kbase, OUT_T)])

    bufs = (rows_a, rows_b)
    desc = pltpu.async_copy(
        x_ref.at[b].at[idxmy.at[pl.ds(0, CH)]], bufs[0], gsem)
    for t in range(OUT_T // CH):
        desc.wait()
        cur = bufs[t % 2]
        if t < OUT_T // CH - 1:
            desc = pltpu.async_copy(
                x_ref.at[b].at[idxmy.at[pl.ds((t + 1) * CH, CH)]],
                bufs[(t + 1) % 2], gsem)
        pltpu.sync_copy(cur, xs_ref.at[b, pl.ds(kbase + t * CH, CH), :])


def _sc_scratch():
    return [
        pltpu.VMEM((L,), jnp.float32),
        pltpu.VMEM((128,), jnp.float32),
        pltpu.VMEM((128,), jnp.int32),
        pltpu.VMEM((SHARD // 128, 128), jnp.int32),
        pltpu.VMEM((SHARD // 128, 128), jnp.int32),
        pltpu.VMEM((OUT_T,), jnp.int32),
        pltpu.VMEM((CH, C), jnp.float32),
        pltpu.VMEM((CH, C), jnp.float32),
        pltpu.VMEM_SHARED((STAGE_ROW,), jnp.int32),
        pltpu.SemaphoreType.DMA,
    ]


def _sc_half0(x, keys, tval, rval):
    mesh = plsc.VectorSubcoreMesh(core_axis_name="c", subcore_axis_name="s")

    def body(*refs):
        _sc_half_body(0, *refs)

    return pl.kernel(
        body,
        (jax.ShapeDtypeStruct((B, K, C), jnp.float32),
         jax.ShapeDtypeStruct((B, K), jnp.int32)),
        mesh=mesh,
        compiler_params=pltpu.CompilerParams(needs_layout_passes=False),
        scratch_types=_sc_scratch(),
    )(x, keys, tval, rval)


def _sc_half1(x, keys, tval, rval, xs_r, idx_r):
    mesh = plsc.VectorSubcoreMesh(core_axis_name="c", subcore_axis_name="s")

    def body(*refs):
        _sc_half_body(2, *refs)

    pl.kernel(
        body,
        (),
        mesh=mesh,
        compiler_params=pltpu.CompilerParams(needs_layout_passes=False),
        scratch_types=_sc_scratch(),
    )(x, keys, tval, rval, xs_r, idx_r)


def kernel(x):
    k0, t0, r0 = _tc_stage(x, 0)
    k1, t1, r1 = _tc_stage(x, HB)
    xs_half, idx_half = _sc_half0(x, k0, t0, r0)
    xs_r = jax.new_ref(xs_half)
    idx_r = jax.new_ref(idx_half)
    _sc_half1(x, k1, t1, r1, xs_r, idx_r)
    return (xs_r[...], idx_r[...])


# speedup vs baseline: 1.6241x; 1.0554x over previous
import jax
import jax.numpy as jnp
from jax import lax
from jax.experimental import pallas as pl
from jax.experimental.pallas import tpu as pltpu
from jax.experimental.pallas import tpu_sc as plsc

B, L, C = 4, 8192, 768
K = L // 2
HB = 2
NSH = 16
SHARD = L // NSH
OUT_T = K // NSH
CH = 64
STAGE_ROW = K + 128


def _tc_body(x_ref, keys_ref, tval_ref, rval_ref):
    x = x_ref[...]
    n = jnp.sqrt(jnp.sum(x * x, axis=2))
    keys_ref[...] = n[:, None, :]
    kb = lax.bitcast_convert_type(keys_ref[...], jnp.int32)

    def body(i, t):
        cand = t | (jnp.int32(1) << (30 - i))
        cnt = jnp.sum((kb >= cand).astype(jnp.int32))
        return jnp.where(cnt >= K, cand, t)

    t_bits = lax.fori_loop(0, 31, body, jnp.int32(0))
    cgt = jnp.sum((kb > t_bits).astype(jnp.int32))
    tval_ref[...] = jnp.full(
        (1, 1, 128), lax.bitcast_convert_type(t_bits, jnp.float32))
    rval_ref[...] = jnp.full((1, 1, 128), K - cgt, jnp.int32)


def _tc_stage(x, bbase):
    keys, tval, rval = pl.pallas_call(
        _tc_body,
        grid=(HB,),
        in_specs=[pl.BlockSpec((1, L, C), lambda b: (bbase + b, 0, 0))],
        out_specs=[
            pl.BlockSpec((1, 1, L), lambda b: (b, 0, 0)),
            pl.BlockSpec((1, 1, 128), lambda b: (b, 0, 0)),
            pl.BlockSpec((1, 1, 128), lambda b: (b, 0, 0)),
        ],
        out_shape=[
            jax.ShapeDtypeStruct((HB, 1, L), jnp.float32),
            jax.ShapeDtypeStruct((HB, 1, 128), jnp.float32),
            jax.ShapeDtypeStruct((HB, 1, 128), jnp.int32),
        ],
    )(x)
    return keys, tval, rval


def _sc_half_body(bbase, x_ref, keys_ref, tval_ref, rval_ref,
                  xs_ref, idx_ref,
                  keys_s, tv, rv, posb, valb, idxmy, rows_a, rows_b,
                  stage_sh, counts_sh, cnt_v, gsem):
    c = lax.axis_index("c")
    s = lax.axis_index("s")
    b = bbase + c
    start = s * SHARD

    pltpu.sync_copy(keys_ref.at[c, 0, pl.ds(start, SHARD)], keys_s)
    pltpu.sync_copy(tval_ref.at[c, 0], tv)
    pltpu.sync_copy(rval_ref.at[c, 0], rv)
    tvec = tv[pl.ds(0, 16)]
    rvec = rv[pl.ds(0, 16)]
    lane = lax.iota(jnp.int32, 16)
    zeros = jnp.zeros((16,), jnp.int32)

    gt = zeros
    eq = zeros
    for j in range(SHARD // 16):
        kv = keys_s[pl.ds(j * 16, 16)]
        gt = gt + jnp.sum((kv > tvec).astype(jnp.int32))
        eq = eq + jnp.sum((kv == tvec).astype(jnp.int32))
    cnt_v[pl.ds(0, 16)] = gt
    cnt_v[pl.ds(16, 16)] = eq
    pltpu.sync_copy(cnt_v.at[pl.ds(0, 16)], counts_sh.at[pl.ds(s * 16, 16)])
    pltpu.sync_copy(cnt_v.at[pl.ds(16, 16)],
                    counts_sh.at[pl.ds(256 + s * 16, 16)])
    plsc.subcore_barrier()
    pltpu.sync_copy(counts_sh, cnt_v)
    gt_all = plsc.load_gather(cnt_v, [lane * 16])
    eq_all = plsc.load_gather(cnt_v, [256 + lane * 16])
    gtp = plsc.cumsum(gt_all) - gt_all
    eqp = plsc.cumsum(eq_all) - eq_all
    cnt_v[pl.ds(0, 16)] = gtp
    cnt_v[pl.ds(16, 16)] = eqp
    svec = zeros + s
    gt0 = plsc.load_gather(cnt_v, [svec])
    eq0 = plsc.load_gather(cnt_v, [16 + svec])

    off = gt0 + jnp.minimum(eq0, rvec)
    eqs = eq0
    for j in range(SHARD // 16):
        kv = keys_s[pl.ds(j * 16, 16)]
        gtm = kv > tvec
        eqm = kv == tvec
        eqi = eqm.astype(jnp.int32)
        eq_excl = plsc.cumsum(eqi) - eqi
        sel = gtm | (eqm & ((eqs + eq_excl) < rvec))
        seli = sel.astype(jnp.int32)
        rank = plsc.cumsum(seli) - seli
        dump = K + ((s * 16 + lane) % 128)
        pos = jnp.where(sel, off + rank, dump)
        posb[j // 8, pl.ds((j % 8) * 16, 16)] = pos
        valb[j // 8, pl.ds((j % 8) * 16, 16)] = start + j * 16 + lane
        off = off + jnp.sum(seli)
        eqs = eqs + jnp.sum(eqi)

    for t in range(SHARD // 128):
        pltpu.sync_copy(valb.at[t], stage_sh.at[posb.at[t]])

    plsc.subcore_barrier()

    kbase = s * OUT_T
    pltpu.sync_copy(stage_sh.at[pl.ds(kbase, OUT_T)], idxmy)
    pltpu.sync_copy(idxmy, idx_ref.at[b, pl.ds(kbase, OUT_T)])

    bufs = (rows_a, rows_b)
    desc = pltpu.async_copy(
        x_ref.at[b].at[idxmy.at[pl.ds(0, CH)]], bufs[0], gsem)
    for t in range(OUT_T // CH):
        desc.wait()
        cur = bufs[t % 2]
        if t < OUT_T // CH - 1:
            desc = pltpu.async_copy(
                x_ref.at[b].at[idxmy.at[pl.ds((t + 1) * CH, CH)]],
                bufs[(t + 1) % 2], gsem)
        pltpu.sync_copy(cur, xs_ref.at[b, pl.ds(kbase + t * CH, CH), :])


def _sc_scratch():
    return [
        pltpu.VMEM((SHARD,), jnp.float32),
        pltpu.VMEM((128,), jnp.float32),
        pltpu.VMEM((128,), jnp.int32),
        pltpu.VMEM((SHARD // 128, 128), jnp.int32),
        pltpu.VMEM((SHARD // 128, 128), jnp.int32),
        pltpu.VMEM((OUT_T,), jnp.int32),
        pltpu.VMEM((CH, C), jnp.float32),
        pltpu.VMEM((CH, C), jnp.float32),
        pltpu.VMEM_SHARED((STAGE_ROW,), jnp.int32),
        pltpu.VMEM_SHARED((512,), jnp.int32),
        pltpu.VMEM((512,), jnp.int32),
        pltpu.SemaphoreType.DMA,
    ]


def _sc_half0(x, keys, tval, rval):
    mesh = plsc.VectorSubcoreMesh(core_axis_name="c", subcore_axis_name="s")

    def body(*refs):
        _sc_half_body(0, *refs)

    return pl.kernel(
        body,
        (jax.ShapeDtypeStruct((B, K, C), jnp.float32),
         jax.ShapeDtypeStruct((B, K), jnp.int32)),
        mesh=mesh,
        compiler_params=pltpu.CompilerParams(needs_layout_passes=False),
        scratch_types=_sc_scratch(),
    )(x, keys, tval, rval)


def _sc_half1(x, keys, tval, rval, xs_r, idx_r):
    mesh = plsc.VectorSubcoreMesh(core_axis_name="c", subcore_axis_name="s")

    def body(*refs):
        _sc_half_body(2, *refs)

    pl.kernel(
        body,
        (),
        mesh=mesh,
        compiler_params=pltpu.CompilerParams(needs_layout_passes=False),
        scratch_types=_sc_scratch(),
    )(x, keys, tval, rval, xs_r, idx_r)


def kernel(x):
    k0, t0, r0 = _tc_stage(x, 0)
    k1, t1, r1 = _tc_stage(x, HB)
    xs_half, idx_half = _sc_half0(x, k0, t0, r0)
    xs_r = jax.new_ref(xs_half)
    idx_r = jax.new_ref(idx_half)
    _sc_half1(x, k1, t1, r1, xs_r, idx_r)
    return (xs_r[...], idx_r[...])
